# baseline (speedup 1.0000x reference)
import jax
import jax.numpy as jnp
from jax.experimental import pallas as pl
from jax.experimental.pallas import tpu as pltpu

_BM = 80
_AB = 400


def _gcn_kernel(adj_ref, x_ref, w1_ref, b1_ref, w2_ref, b2_ref, wd_ref,
                score_ref, s1t_ref, s2_ref, h2_ref):
    p = pl.program_id(0)
    i = pl.program_id(1)
    rows = pl.ds(i * _BM, _BM)
    sub = pl.ds((i % 5) * _BM, _BM)

    @pl.when(jnp.logical_and(p == 0, i == 0))
    def _():
        s1t_ref[...] = jnp.dot(x_ref[...], w1_ref[...],
                               preferred_element_type=jnp.float32).T

    @pl.when(p == 0)
    def _():
        h = jax.lax.dot_general(
            adj_ref[sub, :], s1t_ref[...],
            dimension_numbers=(((1,), (1,)), ((), ())),
            preferred_element_type=jnp.float32)
        h = jnp.maximum(h + b1_ref[...], 0.0)
        s2_ref[rows, :] = jnp.dot(h, w2_ref[...],
                                  preferred_element_type=jnp.float32)

    @pl.when(p == 1)
    def _():
        h2 = jnp.dot(adj_ref[sub, :], s2_ref[...],
                     preferred_element_type=jnp.float32)
        h2_ref[rows, :] = jnp.maximum(h2 + b2_ref[...], 0.0)

    @pl.when(p == 2)
    def _():
        g = jnp.dot(h2_ref[rows, :], wd_ref[...],
                    preferred_element_type=jnp.float32)
        score_ref[...] = jax.lax.dot_general(
            g, h2_ref[...],
            dimension_numbers=(((1,), (1,)), ((), ())),
            preferred_element_type=jnp.float32)


def kernel(adj, x, W1, b1, W2, b2, Wd):
    n, n_feat = x.shape
    n_hid = W1.shape[1]
    n_out = W2.shape[1]
    nb = n // _BM
    nab = n // _AB

    score = pl.pallas_call(
        _gcn_kernel,
        grid=(3, nb),
        in_specs=[
            pl.BlockSpec((_AB, n),
                         lambda p, i: (jnp.where(p == 2, nab - 1, i // 5), 0)),
            pl.BlockSpec((n, n_feat), lambda p, i: (0, 0)),
            pl.BlockSpec((n_feat, n_hid), lambda p, i: (0, 0)),
            pl.BlockSpec((1, n_hid), lambda p, i: (0, 0)),
            pl.BlockSpec((n_hid, n_out), lambda p, i: (0, 0)),
            pl.BlockSpec((1, n_out), lambda p, i: (0, 0)),
            pl.BlockSpec((n_out, n_out), lambda p, i: (0, 0)),
        ],
        out_specs=pl.BlockSpec((_BM, n),
                               lambda p, i: (jnp.where(p == 2, i, 0), 0)),
        out_shape=jax.ShapeDtypeStruct((n, n), jnp.float32),
        scratch_shapes=[
            pltpu.VMEM((n_hid, n), jnp.float32),
            pltpu.VMEM((n, n_out), jnp.float32),
            pltpu.VMEM((n, n_out), jnp.float32),
        ],
        compiler_params=pltpu.CompilerParams(
            vmem_limit_bytes=110 * 1024 * 1024,
        ),
    )(adj, x, W1, b1.reshape(1, n_hid), W2, b2.reshape(1, n_out), Wd)

    return score


# speedup vs baseline: 1.6216x; 1.6216x over previous
import jax
import jax.numpy as jnp
from jax.experimental import pallas as pl
from jax.experimental.pallas import tpu as pltpu

_BM = 200


def _gcn_kernel(adj_ref, x_ref, w1_ref, b1_ref, w2_ref, b2_ref, wd_ref,
                score_ref, s1_ref, s2_ref, h2_ref, g_ref):
    p = pl.program_id(0)
    i = pl.program_id(1)
    rows = pl.ds(i * _BM, _BM)

    @pl.when(jnp.logical_and(p == 0, i == 0))
    def _():
        s1_ref[...] = jnp.dot(x_ref[...], w1_ref[...],
                              preferred_element_type=jnp.float32)

    @pl.when(p == 0)
    def _():
        h = jnp.dot(adj_ref[...], s1_ref[...],
                    preferred_element_type=jnp.float32)
        h = jnp.maximum(h + b1_ref[...], 0.0)
        s2_ref[rows, :] = jnp.dot(h, w2_ref[...],
                                  preferred_element_type=jnp.float32)

    @pl.when(p == 1)
    def _():
        h2 = jnp.dot(adj_ref[...], s2_ref[...],
                     preferred_element_type=jnp.float32)
        h2 = jnp.maximum(h2 + b2_ref[...], 0.0)
        h2_ref[rows, :] = h2
        g_ref[rows, :] = jnp.dot(h2, wd_ref[...],
                                 preferred_element_type=jnp.float32)

    @pl.when(p == 2)
    def _():
        score_ref[...] = jax.lax.dot_general(
            g_ref[rows, :], h2_ref[...],
            dimension_numbers=(((1,), (1,)), ((), ())),
            preferred_element_type=jnp.float32)


def kernel(adj, x, W1, b1, W2, b2, Wd):
    n, n_feat = x.shape
    n_hid = W1.shape[1]
    n_out = W2.shape[1]
    nb = n // _BM

    score = pl.pallas_call(
        _gcn_kernel,
        grid=(3, nb),
        in_specs=[
            pl.BlockSpec((_BM, n),
                         lambda p, i: (jnp.where(p == 2, nb - 1, i), 0)),
            pl.BlockSpec((n, n_feat), lambda p, i: (0, 0)),
            pl.BlockSpec((n_feat, n_hid), lambda p, i: (0, 0)),
            pl.BlockSpec((1, n_hid), lambda p, i: (0, 0)),
            pl.BlockSpec((n_hid, n_out), lambda p, i: (0, 0)),
            pl.BlockSpec((1, n_out), lambda p, i: (0, 0)),
            pl.BlockSpec((n_out, n_out), lambda p, i: (0, 0)),
        ],
        out_specs=pl.BlockSpec((_BM, n),
                               lambda p, i: (jnp.where(p == 2, i, 0), 0)),
        out_shape=jax.ShapeDtypeStruct((n, n), jnp.float32),
        scratch_shapes=[
            pltpu.VMEM((n, n_hid), jnp.float32),
            pltpu.VMEM((n, n_out), jnp.float32),
            pltpu.VMEM((n, n_out), jnp.float32),
            pltpu.VMEM((n, n_out), jnp.float32),
        ],
        compiler_params=pltpu.CompilerParams(
            vmem_limit_bytes=110 * 1024 * 1024,
        ),
    )(adj, x, W1, b1.reshape(1, n_hid), W2, b2.reshape(1, n_out), Wd)

    return score
